# baseline (speedup 1.0000x reference)
import functools

import jax
import jax.numpy as jnp
from jax import lax
from jax.experimental import pallas as pl
from jax.experimental.pallas import tpu as pltpu
from jax.experimental.pallas import tpu_sc as plsc

N = 10000
E = 320000
IN = 128
H = 64
C = 16
ALPHA = 0.1
NITER = 10

NC = 2
NS = 16
NT = NC * NS
NPAD = 10240
HALF = NPAD // 2
RPU = NPAD // NT
AIS = NPAD // NS
CHUNK = 128
K = 10
EPT = E // NT
NCHUNK = 80
EPT_PAD = NCHUNK * CHUNK
PAD = EPT_PAD * NT - E

_MESH = plsc.VectorSubcoreMesh(core_axis_name="c", subcore_axis_name="s")
_SEMMESH = plsc.VectorSubcoreMesh(core_axis_name="cc", subcore_axis_name="ss")


def _mlp_body(x_ref, w1t_ref, w2t_ref, out_ref):
    h = jnp.tanh(jnp.dot(x_ref[...], w1t_ref[...],
                         preferred_element_type=jnp.float32))
    out_ref[...] = jnp.dot(h, w2t_ref[...],
                           preferred_element_type=jnp.float32)


def _mlp(local_preds, w1t, w2t):
    return pl.pallas_call(
        _mlp_body,
        out_shape=jax.ShapeDtypeStruct((NPAD, C), jnp.float32),
    )(local_preds, w1t, w2t)


def _rsqrt(x):
    i = plsc.bitcast(x, jnp.int32)
    i = jnp.int32(0x5F3759DF) - lax.shift_right_arithmetic(i, 1)
    y = plsc.bitcast(i, jnp.float32)
    for _ in range(3):
        y = y * (1.5 - 0.5 * x * y * y)
    return y


def _sc_body(logits_hbm, colb_hbm, rowb_hbm, preds_hbm, u_hbm, pexp_hbm,
             acc, ucache, colv, rowv, gbufa, gbufb, ones, zbuf, dinvv, alv,
             ubuf, gsema, gsemb, ssema, ssemb, xsem):
    cid = lax.axis_index("c")
    sid = lax.axis_index("s")
    w = cid * NS + sid
    ubase = cid * HALF + sid * RPU
    xbase = (1 - cid) * HALF + sid * RPU

    def fire_gathers(c0, gbuf, gsem):
        for j in range(K):
            pltpu.async_copy(ucache.at[colv.at[c0 + j]],
                             gbuf.at[pl.ds(j * CHUNK, CHUNK)], gsem)

    def drain_gathers(c0, gbuf, gsem):
        for j in range(K):
            pltpu.make_async_copy(ucache.at[colv.at[c0 + j]],
                                  gbuf.at[pl.ds(j * CHUNK, CHUNK)],
                                  gsem).wait()

    def refresh_ucache():
        pltpu.sync_copy(ubuf, ucache.at[pl.ds(ubase, RPU)])
        pltpu.sync_copy(u_hbm.at[pl.ds(xbase, RPU)], xbuf)
        pltpu.sync_copy(xbuf, ucache.at[pl.ds(xbase, RPU)])

    def fire_scatters(c0, gbuf, ssem):
        for j in range(K):
            pltpu.async_copy(gbuf.at[pl.ds(j * CHUNK, CHUNK)],
                             acc.at[rowv.at[c0 + j]], ssem, add=True)

    def drain_scatters(c0, gbuf, ssem):
        for j in range(K):
            pltpu.make_async_copy(gbuf.at[pl.ds(j * CHUNK, CHUNK)],
                                  acc.at[rowv.at[c0 + j]], ssem).wait()

    def export_and_rendezvous():
        stage = gbufb.at[pl.ds(RPU, RPU)]
        pltpu.sync_copy(acc.at[pl.ds(xbase, RPU)], stage)
        pltpu.sync_copy(zbuf.at[pl.ds(0, RPU)], acc.at[pl.ds(xbase, RPU)])
        pltpu.sync_copy(stage, pexp_hbm.at[pl.ds(xbase, RPU)])
        pltpu.semaphore_signal(xsem, 1, device_id={"cc": 1 - cid, "ss": sid})
        pltpu.semaphore_wait(xsem, 1)

    def full_barrier():
        plsc.subcore_barrier()
        pltpu.semaphore_signal(xsem, 1, device_id={"cc": 1 - cid, "ss": sid})
        pltpu.semaphore_wait(xsem, 1)

    tbuf = gbufa.at[pl.ds(0, RPU)]
    xbuf = gbufa.at[pl.ds(RPU, RPU)]
    pbuf = gbufb.at[pl.ds(0, RPU)]

    pltpu.sync_copy(colb_hbm.at[w], colv)
    pltpu.sync_copy(rowb_hbm.at[w], rowv)

    @pl.loop(0, CHUNK)
    def _(j):
        ones[j] = jnp.full((C,), 1.0, jnp.float32)

    @pl.loop(0, AIS)
    def _(j):
        zbuf[j] = jnp.full((C,), 0.0, jnp.float32)

    pltpu.sync_copy(zbuf, acc.at[pl.ds(sid * AIS, AIS)])
    plsc.subcore_barrier()

    @pl.loop(0, NCHUNK, step=2 * K)
    def _(c0):
        for j in range(K):
            pltpu.async_copy(ones, acc.at[rowv.at[c0 + j]], ssema, add=True)
        for j in range(K):
            pltpu.async_copy(ones, acc.at[rowv.at[c0 + K + j]], ssemb,
                             add=True)
        for j in range(K):
            pltpu.make_async_copy(ones, acc.at[rowv.at[c0 + j]],
                                  ssema).wait()
        for j in range(K):
            pltpu.make_async_copy(ones, acc.at[rowv.at[c0 + K + j]],
                                  ssemb).wait()

    plsc.subcore_barrier()
    export_and_rendezvous()

    pltpu.sync_copy(acc.at[pl.ds(ubase, RPU)], tbuf)
    pltpu.sync_copy(zbuf.at[pl.ds(0, RPU)], acc.at[pl.ds(ubase, RPU)])
    pltpu.sync_copy(pexp_hbm.at[pl.ds(ubase, RPU)], xbuf)
    pltpu.sync_copy(logits_hbm.at[pl.ds(ubase, RPU)], alv)

    @pl.loop(0, RPU)
    def _(j):
        d = _rsqrt(tbuf[j] + xbuf[j] + 1.0)
        dinvv[j] = d
        lg = alv[j]
        ubuf[j] = d * lg
        alv[j] = ALPHA * lg

    pltpu.sync_copy(ubuf, u_hbm.at[pl.ds(ubase, RPU)])
    full_barrier()
    refresh_ucache()
    plsc.subcore_barrier()

    @pl.loop(0, NITER)
    def _(it):
        fire_gathers(0, gbufa, gsema)

        @pl.loop(0, NCHUNK, step=2 * K)
        def _(c0):
            fire_gathers(c0 + K, gbufb, gsemb)
            drain_gathers(c0, gbufa, gsema)
            fire_scatters(c0, gbufa, ssema)
            drain_scatters(c0, gbufa, ssema)

            @pl.when(c0 + 2 * K < NCHUNK)
            def _():
                fire_gathers(c0 + 2 * K, gbufa, gsema)

            drain_gathers(c0 + K, gbufb, gsemb)
            fire_scatters(c0 + K, gbufb, ssemb)
            drain_scatters(c0 + K, gbufb, ssemb)

        plsc.subcore_barrier()
        export_and_rendezvous()

        pltpu.sync_copy(acc.at[pl.ds(ubase, RPU)], tbuf)
        pltpu.sync_copy(zbuf.at[pl.ds(0, RPU)], acc.at[pl.ds(ubase, RPU)])
        pltpu.sync_copy(pexp_hbm.at[pl.ds(ubase, RPU)], xbuf)

        @pl.loop(0, RPU)
        def _(j):
            t = tbuf[j] + xbuf[j] + ubuf[j]
            d = dinvv[j]
            p = (1.0 - ALPHA) * d * t + alv[j]
            pbuf[j] = p
            ubuf[j] = d * p

        pltpu.sync_copy(pbuf, preds_hbm.at[pl.ds(ubase, RPU)])
        pltpu.sync_copy(ubuf, u_hbm.at[pl.ds(ubase, RPU)])
        full_barrier()
        refresh_ucache()
        plsc.subcore_barrier()


_sc_kernel = functools.partial(
    pl.kernel,
    out_type=(jax.ShapeDtypeStruct((NPAD, C), jnp.float32),
              jax.ShapeDtypeStruct((NPAD, C), jnp.float32),
              jax.ShapeDtypeStruct((NPAD, C), jnp.float32)),
    mesh=_MESH,
    compiler_params=pltpu.CompilerParams(use_tc_tiling_on_sc=False,
                                         needs_layout_passes=False),
    scratch_types=[
        pltpu.VMEM_SHARED((NPAD, C), jnp.float32),
        pltpu.VMEM_SHARED((NPAD, C), jnp.float32),
        pltpu.VMEM((NCHUNK, CHUNK), jnp.int32),
        pltpu.VMEM((NCHUNK, CHUNK), jnp.int32),
        pltpu.VMEM((K * CHUNK, C), jnp.float32),
        pltpu.VMEM((K * CHUNK, C), jnp.float32),
        pltpu.VMEM((CHUNK, C), jnp.float32),
        pltpu.VMEM((AIS, C), jnp.float32),
        pltpu.VMEM((RPU, C), jnp.float32),
        pltpu.VMEM((RPU, C), jnp.float32),
        pltpu.VMEM((RPU, C), jnp.float32),
        pltpu.SemaphoreType.DMA,
        pltpu.SemaphoreType.DMA,
        pltpu.SemaphoreType.DMA,
        pltpu.SemaphoreType.DMA,
        pltpu.SemaphoreType.REGULAR @ _SEMMESH,
    ],
)(_sc_body)


def kernel(local_preds, W1, W2, edge_row, edge_col):
    xpad = jnp.pad(local_preds, ((0, NPAD - N), (0, 0)))
    logits = _mlp(xpad, W1.T, W2.T)
    colb = jnp.concatenate(
        [edge_col, jnp.zeros((PAD,), jnp.int32)]).reshape(NT, NCHUNK, CHUNK)
    rowb = jnp.concatenate(
        [edge_row, jnp.full((PAD,), N, jnp.int32)]).reshape(NT, NCHUNK, CHUNK)
    preds, _, _ = _sc_kernel(logits, colb, rowb)
    return preds[:N]


# speedup vs baseline: 1.0219x; 1.0219x over previous
import functools

import jax
import jax.numpy as jnp
from jax import lax
from jax.experimental import pallas as pl
from jax.experimental.pallas import tpu as pltpu
from jax.experimental.pallas import tpu_sc as plsc

N = 10000
E = 320000
IN = 128
H = 64
C = 16
ALPHA = 0.1
NITER = 10

NC = 2
NS = 16
NT = NC * NS
NPAD = 10240
HALF = NPAD // 2
RPU = NPAD // NT
AIS = NPAD // NS
CHUNK = 128
K = 10
EPT = E // NT
NCHUNK = 80
EPT_PAD = NCHUNK * CHUNK
PAD = EPT_PAD * NT - E

_MESH = plsc.VectorSubcoreMesh(core_axis_name="c", subcore_axis_name="s")
_SEMMESH = plsc.VectorSubcoreMesh(core_axis_name="cc", subcore_axis_name="ss")


def _mlp_body(x_ref, w1t_ref, w2t_ref, out_ref):
    h = jnp.tanh(jnp.dot(x_ref[...], w1t_ref[...],
                         preferred_element_type=jnp.float32))
    out_ref[...] = jnp.dot(h, w2t_ref[...],
                           preferred_element_type=jnp.float32)


def _mlp(local_preds, w1t, w2t):
    return pl.pallas_call(
        _mlp_body,
        out_shape=jax.ShapeDtypeStruct((NPAD, C), jnp.float32),
    )(local_preds, w1t, w2t)


def _rsqrt(x):
    i = plsc.bitcast(x, jnp.int32)
    i = jnp.int32(0x5F3759DF) - lax.shift_right_arithmetic(i, 1)
    y = plsc.bitcast(i, jnp.float32)
    for _ in range(3):
        y = y * (1.5 - 0.5 * x * y * y)
    return y


def _sc_body(logits_hbm, colb_hbm, rowb_hbm, preds_hbm, u_hbm, pexp_hbm,
             acc, ucache, colv, rowv, gbufa, gbufb, ones, zbuf, dinvv, alv,
             ubuf, gsema, gsemb, ssema, ssemb, xsem):
    cid = lax.axis_index("c")
    sid = lax.axis_index("s")
    w = cid * NS + sid
    ubase = cid * HALF + sid * RPU
    xbase = (1 - cid) * HALF + sid * RPU

    def fire_gathers(c0, gbuf, gsem):
        for j in range(K):
            pltpu.async_copy(ucache.at[colv.at[c0 + j]],
                             gbuf.at[pl.ds(j * CHUNK, CHUNK)], gsem)

    def drain_gathers(c0, gbuf, gsem):
        for j in range(K):
            pltpu.make_async_copy(ucache.at[colv.at[c0 + j]],
                                  gbuf.at[pl.ds(j * CHUNK, CHUNK)],
                                  gsem).wait()

    def refresh_ucache():
        pltpu.sync_copy(ubuf, ucache.at[pl.ds(ubase, RPU)])
        pltpu.sync_copy(u_hbm.at[pl.ds(xbase, RPU)], xbuf)
        pltpu.sync_copy(xbuf, ucache.at[pl.ds(xbase, RPU)])

    def fire_scatters(c0, gbuf, ssem):
        for j in range(K):
            pltpu.async_copy(gbuf.at[pl.ds(j * CHUNK, CHUNK)],
                             acc.at[rowv.at[c0 + j]], ssem, add=True)

    def drain_scatters(c0, gbuf, ssem):
        for j in range(K):
            pltpu.make_async_copy(gbuf.at[pl.ds(j * CHUNK, CHUNK)],
                                  acc.at[rowv.at[c0 + j]], ssem).wait()

    def export_and_rendezvous():
        stage = gbufb.at[pl.ds(RPU, RPU)]
        pltpu.async_copy(acc.at[pl.ds(xbase, RPU)], stage, gsema)
        pltpu.async_copy(acc.at[pl.ds(ubase, RPU)], tbuf, gsemb)
        pltpu.make_async_copy(acc.at[pl.ds(xbase, RPU)], stage, gsema).wait()
        pltpu.async_copy(stage, pexp_hbm.at[pl.ds(xbase, RPU)], gsema)
        pltpu.sync_copy(zbuf.at[pl.ds(0, RPU)], acc.at[pl.ds(xbase, RPU)])
        pltpu.make_async_copy(stage, pexp_hbm.at[pl.ds(xbase, RPU)],
                              gsema).wait()
        pltpu.semaphore_signal(xsem, 1, device_id={"cc": 1 - cid, "ss": sid})
        pltpu.semaphore_wait(xsem, 1)

    def full_barrier():
        plsc.subcore_barrier()
        pltpu.semaphore_signal(xsem, 1, device_id={"cc": 1 - cid, "ss": sid})
        pltpu.semaphore_wait(xsem, 1)

    tbuf = gbufa.at[pl.ds(0, RPU)]
    xbuf = gbufa.at[pl.ds(RPU, RPU)]
    pbuf = gbufb.at[pl.ds(0, RPU)]

    pltpu.sync_copy(colb_hbm.at[w], colv)
    pltpu.sync_copy(rowb_hbm.at[w], rowv)

    @pl.loop(0, CHUNK)
    def _(j):
        ones[j] = jnp.full((C,), 1.0, jnp.float32)

    @pl.loop(0, AIS)
    def _(j):
        zbuf[j] = jnp.full((C,), 0.0, jnp.float32)

    pltpu.sync_copy(zbuf, acc.at[pl.ds(sid * AIS, AIS)])
    plsc.subcore_barrier()

    @pl.loop(0, NCHUNK, step=2 * K)
    def _(c0):
        for j in range(K):
            pltpu.async_copy(ones, acc.at[rowv.at[c0 + j]], ssema, add=True)
        for j in range(K):
            pltpu.async_copy(ones, acc.at[rowv.at[c0 + K + j]], ssemb,
                             add=True)
        for j in range(K):
            pltpu.make_async_copy(ones, acc.at[rowv.at[c0 + j]],
                                  ssema).wait()
        for j in range(K):
            pltpu.make_async_copy(ones, acc.at[rowv.at[c0 + K + j]],
                                  ssemb).wait()

    plsc.subcore_barrier()
    export_and_rendezvous()

    pltpu.make_async_copy(acc.at[pl.ds(ubase, RPU)], tbuf, gsemb).wait()
    pltpu.sync_copy(zbuf.at[pl.ds(0, RPU)], acc.at[pl.ds(ubase, RPU)])
    pltpu.sync_copy(pexp_hbm.at[pl.ds(ubase, RPU)], xbuf)
    pltpu.sync_copy(logits_hbm.at[pl.ds(ubase, RPU)], alv)

    @pl.loop(0, RPU)
    def _(j):
        d = _rsqrt(tbuf[j] + xbuf[j] + 1.0)
        dinvv[j] = d
        lg = alv[j]
        ubuf[j] = d * lg
        alv[j] = ALPHA * lg

    pltpu.sync_copy(ubuf, u_hbm.at[pl.ds(ubase, RPU)])
    full_barrier()
    refresh_ucache()
    plsc.subcore_barrier()

    @pl.loop(0, NITER)
    def _(it):
        fire_gathers(0, gbufa, gsema)

        @pl.loop(0, NCHUNK, step=2 * K)
        def _(c0):
            fire_gathers(c0 + K, gbufb, gsemb)
            drain_gathers(c0, gbufa, gsema)
            fire_scatters(c0, gbufa, ssema)
            drain_scatters(c0, gbufa, ssema)

            @pl.when(c0 + 2 * K < NCHUNK)
            def _():
                fire_gathers(c0 + 2 * K, gbufa, gsema)

            drain_gathers(c0 + K, gbufb, gsemb)
            fire_scatters(c0 + K, gbufb, ssemb)
            drain_scatters(c0 + K, gbufb, ssemb)

        plsc.subcore_barrier()
        export_and_rendezvous()

        pltpu.make_async_copy(acc.at[pl.ds(ubase, RPU)], tbuf, gsemb).wait()
        pltpu.async_copy(zbuf.at[pl.ds(0, RPU)], acc.at[pl.ds(ubase, RPU)],
                         ssema)
        pltpu.sync_copy(pexp_hbm.at[pl.ds(ubase, RPU)], xbuf)

        @pl.loop(0, RPU)
        def _(j):
            t = tbuf[j] + xbuf[j] + ubuf[j]
            d = dinvv[j]
            p = (1.0 - ALPHA) * d * t + alv[j]
            pbuf[j] = p
            ubuf[j] = d * p

        pltpu.async_copy(pbuf, preds_hbm.at[pl.ds(ubase, RPU)], gsema)
        pltpu.async_copy(ubuf, u_hbm.at[pl.ds(ubase, RPU)], gsemb)
        pltpu.async_copy(ubuf, ucache.at[pl.ds(ubase, RPU)], ssemb)
        pltpu.make_async_copy(zbuf.at[pl.ds(0, RPU)],
                              acc.at[pl.ds(ubase, RPU)], ssema).wait()
        pltpu.make_async_copy(pbuf, preds_hbm.at[pl.ds(ubase, RPU)],
                              gsema).wait()
        pltpu.make_async_copy(ubuf, u_hbm.at[pl.ds(ubase, RPU)],
                              gsemb).wait()
        pltpu.make_async_copy(ubuf, ucache.at[pl.ds(ubase, RPU)],
                              ssemb).wait()
        full_barrier()
        pltpu.sync_copy(u_hbm.at[pl.ds(xbase, RPU)], xbuf)
        pltpu.sync_copy(xbuf, ucache.at[pl.ds(xbase, RPU)])
        plsc.subcore_barrier()


_sc_kernel = functools.partial(
    pl.kernel,
    out_type=(jax.ShapeDtypeStruct((NPAD, C), jnp.float32),
              jax.ShapeDtypeStruct((NPAD, C), jnp.float32),
              jax.ShapeDtypeStruct((NPAD, C), jnp.float32)),
    mesh=_MESH,
    compiler_params=pltpu.CompilerParams(use_tc_tiling_on_sc=False,
                                         needs_layout_passes=False),
    scratch_types=[
        pltpu.VMEM_SHARED((NPAD, C), jnp.float32),
        pltpu.VMEM_SHARED((NPAD, C), jnp.float32),
        pltpu.VMEM((NCHUNK, CHUNK), jnp.int32),
        pltpu.VMEM((NCHUNK, CHUNK), jnp.int32),
        pltpu.VMEM((K * CHUNK, C), jnp.float32),
        pltpu.VMEM((K * CHUNK, C), jnp.float32),
        pltpu.VMEM((CHUNK, C), jnp.float32),
        pltpu.VMEM((AIS, C), jnp.float32),
        pltpu.VMEM((RPU, C), jnp.float32),
        pltpu.VMEM((RPU, C), jnp.float32),
        pltpu.VMEM((RPU, C), jnp.float32),
        pltpu.SemaphoreType.DMA,
        pltpu.SemaphoreType.DMA,
        pltpu.SemaphoreType.DMA,
        pltpu.SemaphoreType.DMA,
        pltpu.SemaphoreType.REGULAR @ _SEMMESH,
    ],
)(_sc_body)


def kernel(local_preds, W1, W2, edge_row, edge_col):
    xpad = jnp.pad(local_preds, ((0, NPAD - N), (0, 0)))
    logits = _mlp(xpad, W1.T, W2.T)
    colb = jnp.concatenate(
        [edge_col, jnp.zeros((PAD,), jnp.int32)]).reshape(NT, NCHUNK, CHUNK)
    rowb = jnp.concatenate(
        [edge_row, jnp.full((PAD,), N, jnp.int32)]).reshape(NT, NCHUNK, CHUNK)
    preds, _, _ = _sc_kernel(logits, colb, rowb)
    return preds[:N]
